# Initial kernel scaffold; baseline (speedup 1.0000x reference)
import jax
import jax.numpy as jnp
from jax.experimental import pallas as pl


def kernel(x, emb_s, emb_t):
    raise NotImplementedError("write your pallas kernel here")


# speedup vs baseline: 8.5132x; 8.5132x over previous
import functools

import jax
import jax.numpy as jnp
from jax.experimental import pallas as pl
from jax.experimental.pallas import tpu as pltpu

_TOPK = 4096
_CH = 256
_RB = 256


def _keys_for_rows(s_rows, t, row0, n):
    ch = s_rows.shape[0]
    blk = s_rows[:, None] * t[None, :]
    rows = jax.lax.broadcasted_iota(jnp.int32, (ch, n), 0) + row0
    cols = jax.lax.broadcasted_iota(jnp.int32, (ch, n), 1)
    blk = jnp.where(rows == cols, -jnp.inf, blk)
    i = jax.lax.bitcast_convert_type(blk, jnp.int32)
    mask = jax.lax.shift_right_arithmetic(i, 31) & jnp.int32(0x7FFFFFFF)
    return i ^ mask, rows * n + cols


def _threshold_kernel(s_ref, t_ref, thr_ref, keys_ref, *, n, topk):
    t = t_ref[0, 0, :]
    nch = n // _CH

    def fill(c, _):
        srows = s_ref[0, 0, pl.ds(c * _CH, _CH)]
        keys, _flat = _keys_for_rows(srows, t, c * _CH, n)
        keys_ref[pl.ds(c * _CH, _CH), :] = keys
        return 0

    jax.lax.fori_loop(0, nch, fill, 0)

    def count_cmp(c, strict):
        def body(ch, acc):
            blk = keys_ref[pl.ds(ch * _CH, _CH), :]
            cond = (blk > c) if strict else (blk >= c)
            return acc + jnp.sum(cond.astype(jnp.int32))
        return jax.lax.fori_loop(0, nch, body, jnp.int32(0))

    def value_step(it, c):
        trial = c + jax.lax.shift_left(jnp.int32(1), 31 - it)
        cnt = count_cmp(trial, False)
        return jnp.where(cnt >= topk, trial, c)

    v_k = jax.lax.fori_loop(0, 32, value_step, jnp.int32(-2147483648))

    n_gt = count_cmp(v_k, True)
    r = jnp.int32(topk) - n_gt

    def count_tie(m):
        def body(ch, acc):
            blk = keys_ref[pl.ds(ch * _CH, _CH), :]
            rows = jax.lax.broadcasted_iota(jnp.int32, (_CH, n), 0) + ch * _CH
            cols = jax.lax.broadcasted_iota(jnp.int32, (_CH, n), 1)
            flat = rows * n + cols
            return acc + jnp.sum(((blk == v_k) & (flat <= m)).astype(jnp.int32))
        return jax.lax.fori_loop(0, nch, body, jnp.int32(0))

    def tie_step(_, lohi):
        lo, hi = lohi
        mid = lo + ((hi - lo) >> 1)
        ok = count_tie(mid) >= r
        return jnp.where(ok, lo, mid + 1), jnp.where(ok, mid, hi)

    lo, hi = jax.lax.fori_loop(
        0, 22, tie_step, (jnp.int32(0), jnp.int32(n * n - 1)))

    lane = jax.lax.broadcasted_iota(jnp.int32, (1, 128), 1)
    thr_ref[0, 0, :] = jnp.where(lane == 0, v_k,
                                 jnp.where(lane == 1, lo, 0))[0, :]


def _write_kernel(s_ref, t_ref, thr_ref, out_ref, *, n):
    rb = pl.program_id(1)
    v_k = thr_ref[0, 0, 0]
    icut = thr_ref[0, 0, 1]
    s_rows = s_ref[0, 0, :]
    keys, flat = _keys_for_rows(s_rows, t_ref[0, 0, :], rb * _RB, n)
    sel = (keys > v_k) | ((keys == v_k) & (flat <= icut))
    out_ref[0, :, :] = sel.astype(jnp.float32)


def kernel(x, emb_s, emb_t):
    del x
    b, n = emb_s.shape[0], emb_s.shape[1]
    s = emb_s.reshape(b, 1, n)
    t = emb_t.reshape(b, 1, n)

    thr = pl.pallas_call(
        functools.partial(_threshold_kernel, n=n, topk=_TOPK),
        grid=(b,),
        in_specs=[
            pl.BlockSpec((1, 1, n), lambda i: (i, 0, 0)),
            pl.BlockSpec((1, 1, n), lambda i: (i, 0, 0)),
        ],
        out_specs=pl.BlockSpec((1, 1, 128), lambda i: (i, 0, 0)),
        out_shape=jax.ShapeDtypeStruct((b, 1, 128), jnp.int32),
        scratch_shapes=[pltpu.VMEM((n, n), jnp.int32)],
    )(s, t)

    out = pl.pallas_call(
        functools.partial(_write_kernel, n=n),
        grid=(b, n // _RB),
        in_specs=[
            pl.BlockSpec((1, 1, _RB), lambda i, j: (i, 0, j)),
            pl.BlockSpec((1, 1, n), lambda i, j: (i, 0, 0)),
            pl.BlockSpec((1, 1, 128), lambda i, j: (i, 0, 0)),
        ],
        out_specs=pl.BlockSpec((1, _RB, n), lambda i, j: (i, j, 0)),
        out_shape=jax.ShapeDtypeStruct((b, n, n), jnp.float32),
    )(s, t, thr)

    return out
